# Initial kernel scaffold; baseline (speedup 1.0000x reference)
import jax
import jax.numpy as jnp
from jax.experimental import pallas as pl


def kernel(u, v, r, n, c, u_emb_w, v_emb_w, Wu1, Wv1, bu1, bv1, Wu2, Wv2, bu2, bv2, Q):
    raise NotImplementedError("write your pallas kernel here")


# speedup vs baseline: 1.7141x; 1.7141x over previous
import jax
import jax.numpy as jnp
from jax.experimental import pallas as pl
from jax.experimental.pallas import tpu as pltpu

_NU, _NV, _NC, _D, _H = 943, 1682, 5, 64, 32
_BU, _BV = 320, 256
_GU = (_NU + _BU - 1) // _BU
_GV = (_NV + _BV - 1) // _BV


def _gconv_kernel(r_ref, c_ref, uf_ref, vf_ref, wu_ref, wv_ref, bu_ref, bv_ref,
                  u2_ref, v2_ref, code_ref):
    i = pl.program_id(0)
    j = pl.program_id(1)

    rows = jax.lax.broadcasted_iota(jnp.int32, (_BU, 1), 0) + i * _BU
    cols = jax.lax.broadcasted_iota(jnp.int32, (1, _BV), 1) + j * _BV
    row_ok = rows < _NU
    col_ok = cols < _NV
    valid = jnp.logical_and(row_ok, col_ok)

    cblk = jnp.where(valid, c_ref[...], 0.0)
    ufb = jnp.where(row_ok, uf_ref[...], 0.0)
    colsT = jax.lax.broadcasted_iota(jnp.int32, (_BV, 1), 0) + j * _BV
    vfb = jnp.where(colsT < _NV, vf_ref[...], 0.0)

    rblk = r_ref[...]

    ucontrib = jnp.zeros((_BU, _H), jnp.float32)
    vcontrib = jnp.zeros((_BV, _H), jnp.float32)
    best = jnp.where(valid, rblk[0], 0.0)
    cls = jnp.zeros((_BU, _BV), jnp.int32)
    for k in range(_NC):
        rk = jnp.where(valid, rblk[k], 0.0)
        if k > 0:
            gt = rk > best
            cls = jnp.where(gt, k, cls)
            best = jnp.maximum(best, rk)
        rc = rk * cblk
        t_v = jnp.dot(vfb, wv_ref[k], preferred_element_type=jnp.float32)
        ucontrib = ucontrib + jnp.dot(rc, t_v, preferred_element_type=jnp.float32)
        t_u = jnp.dot(ufb, wu_ref[k], preferred_element_type=jnp.float32)
        vcontrib = vcontrib + jax.lax.dot_general(
            rc, t_u, (((0,), (0,)), ((), ())),
            preferred_element_type=jnp.float32)

    code_ref[...] = jnp.where(best > 0.0, cls + 1, 0).astype(jnp.int8)

    @pl.when(j == 0)
    def _():
        u2_ref[...] = ucontrib

    @pl.when(j > 0)
    def _():
        u2_ref[...] = u2_ref[...] + ucontrib

    @pl.when(j == _GV - 1)
    def _():
        u2_ref[...] = jnp.maximum(u2_ref[...] + bu_ref[...], 0.0)

    sl = pl.ds(j * _BV, _BV)

    @pl.when(i == 0)
    def _():
        v2_ref[sl, :] = vcontrib

    @pl.when(i > 0)
    def _():
        v2_ref[sl, :] = v2_ref[sl, :] + vcontrib

    @pl.when(i == _GU - 1)
    def _():
        v2_ref[sl, :] = jnp.maximum(v2_ref[sl, :] + bv_ref[...], 0.0)


def _decode_kernel(u2_ref, v2_ref, q_ref, code_ref,
                   out_ref, loss_ref, acc_ref, sums):
    i = pl.program_id(0)
    j = pl.program_id(1)

    @pl.when(jnp.logical_and(i == 0, j == 0))
    def _():
        sums[0] = 0.0
        sums[1] = 0.0
        sums[2] = 0.0

    rows = jax.lax.broadcasted_iota(jnp.int32, (_BU, 1), 0) + i * _BU
    cols = jax.lax.broadcasted_iota(jnp.int32, (1, _BV), 1) + j * _BV
    valid = jnp.logical_and(rows < _NU, cols < _NV)

    u2b = u2_ref[...]
    v2b = v2_ref[...]
    code = code_ref[...].astype(jnp.int32)
    rated = jnp.logical_and(valid, code > 0)
    tcls = code - 1

    zs = []
    for k in range(_NC):
        uq = jnp.dot(u2b, q_ref[k], preferred_element_type=jnp.float32)
        z = jax.lax.dot_general(
            uq, v2b, (((1,), (1,)), ((), ())),
            preferred_element_type=jnp.float32)
        out_ref[k] = z
        zs.append(z)

    m = zs[0]
    pred = jnp.zeros((_BU, _BV), jnp.int32)
    for k in range(1, _NC):
        gt = zs[k] > m
        pred = jnp.where(gt, k, pred)
        m = jnp.maximum(m, zs[k])
    s = jnp.zeros((_BU, _BV), jnp.float32)
    for k in range(_NC):
        s = s + jnp.exp(zs[k] - m)
    lse = m + jnp.log(s)

    ztrue = jnp.zeros((_BU, _BV), jnp.float32)
    for k in range(_NC):
        ztrue = jnp.where(tcls == k, zs[k], ztrue)

    loss_c = jnp.sum(jnp.where(rated, ztrue - lse, 0.0))
    mask_c = jnp.sum(jnp.where(rated, 1.0, 0.0))
    corr_c = jnp.sum(jnp.where(jnp.logical_and(rated, pred == tcls), 1.0, 0.0))
    sums[0] = sums[0] + loss_c
    sums[1] = sums[1] + mask_c
    sums[2] = sums[2] + corr_c

    @pl.when(jnp.logical_and(i == _GU - 1, j == _GV - 1))
    def _():
        denom = jnp.maximum(sums[1], 1.0)
        loss_ref[...] = jnp.full((1, 1), -sums[0] / denom, jnp.float32)
        acc_ref[...] = jnp.full((1, 1), sums[2] / denom, jnp.float32)


def kernel(u, v, r, n, c, u_emb_w, v_emb_w, Wu1, Wv1, bu1, bv1,
           Wu2, Wv2, bu2, bv2, Q):
    uf = jnp.take(u_emb_w, u, axis=0)
    vf = jnp.take(v_emb_w, v, axis=0)

    u2p, v2p, code = pl.pallas_call(
        _gconv_kernel,
        grid=(_GU, _GV),
        in_specs=[
            pl.BlockSpec((_NC, _BU, _BV), lambda i, j: (0, i, j)),
            pl.BlockSpec((_BU, _BV), lambda i, j: (i, j)),
            pl.BlockSpec((_BU, _D), lambda i, j: (i, 0)),
            pl.BlockSpec((_BV, _D), lambda i, j: (j, 0)),
            pl.BlockSpec((_NC, _D, _H), lambda i, j: (0, 0, 0)),
            pl.BlockSpec((_NC, _D, _H), lambda i, j: (0, 0, 0)),
            pl.BlockSpec((1, _H), lambda i, j: (0, 0)),
            pl.BlockSpec((1, _H), lambda i, j: (0, 0)),
        ],
        out_specs=[
            pl.BlockSpec((_BU, _H), lambda i, j: (i, 0)),
            pl.BlockSpec((_GV * _BV, _H), lambda i, j: (0, 0)),
            pl.BlockSpec((_BU, _BV), lambda i, j: (i, j)),
        ],
        out_shape=[
            jax.ShapeDtypeStruct((_GU * _BU, _H), jnp.float32),
            jax.ShapeDtypeStruct((_GV * _BV, _H), jnp.float32),
            jax.ShapeDtypeStruct((_GU * _BU, _GV * _BV), jnp.int8),
        ],
        compiler_params=pltpu.CompilerParams(
            dimension_semantics=("arbitrary", "arbitrary")),
    )(r, c, uf, vf, Wu2, Wv2, bu2.reshape(1, _H), bv2.reshape(1, _H))

    outputs, lossm, accm = pl.pallas_call(
        _decode_kernel,
        grid=(_GU, _GV),
        in_specs=[
            pl.BlockSpec((_BU, _H), lambda i, j: (i, 0)),
            pl.BlockSpec((_BV, _H), lambda i, j: (j, 0)),
            pl.BlockSpec((_NC, _H, _H), lambda i, j: (0, 0, 0)),
            pl.BlockSpec((_BU, _BV), lambda i, j: (i, j)),
        ],
        out_specs=[
            pl.BlockSpec((_NC, _BU, _BV), lambda i, j: (0, i, j)),
            pl.BlockSpec((1, 1), lambda i, j: (0, 0)),
            pl.BlockSpec((1, 1), lambda i, j: (0, 0)),
        ],
        out_shape=[
            jax.ShapeDtypeStruct((_NC, _NU, _NV), jnp.float32),
            jax.ShapeDtypeStruct((1, 1), jnp.float32),
            jax.ShapeDtypeStruct((1, 1), jnp.float32),
        ],
        scratch_shapes=[pltpu.SMEM((4,), jnp.float32)],
        compiler_params=pltpu.CompilerParams(
            dimension_semantics=("arbitrary", "arbitrary")),
    )(u2p, v2p, Q, code)

    return outputs, lossm[0, 0], accm[0, 0]


# speedup vs baseline: 1.7578x; 1.0255x over previous
import jax
import jax.numpy as jnp
from jax.experimental import pallas as pl
from jax.experimental.pallas import tpu as pltpu

_NU, _NV, _NC, _D, _H = 943, 1682, 5, 64, 32
_BU, _BV = 320, 256
_GU = (_NU + _BU - 1) // _BU
_GV = (_NV + _BV - 1) // _BV


def _gconv_kernel(r_ref, c_ref, uf_ref, vf_ref, wu_ref, wv_ref, bu_ref, bv_ref,
                  u2_ref, v2t_ref, code_ref):
    i = pl.program_id(0)
    j = pl.program_id(1)

    rows = jax.lax.broadcasted_iota(jnp.int32, (_BU, 1), 0) + i * _BU
    cols = jax.lax.broadcasted_iota(jnp.int32, (1, _BV), 1) + j * _BV
    row_ok = rows < _NU
    valid = jnp.logical_and(row_ok, cols < _NV)

    cblk = c_ref[...]
    ufb = jnp.where(row_ok, uf_ref[...], 0.0)
    colsT = jax.lax.broadcasted_iota(jnp.int32, (_BV, 1), 0) + j * _BV
    vfb = jnp.where(colsT < _NV, vf_ref[...], 0.0)

    rblk = r_ref[...]

    ucontrib = jnp.zeros((_BU, _H), jnp.float32)
    vcontribT = jnp.zeros((_H, _BV), jnp.float32)
    code_f = jnp.zeros((_BU, _BV), jnp.float32)
    for k in range(_NC):
        rk = rblk[k]
        code_f = code_f + rk * float(k + 1)
        rc = jnp.where(valid, rk * cblk, 0.0)
        t_v = jnp.dot(vfb, wv_ref[k], preferred_element_type=jnp.float32)
        ucontrib = ucontrib + jnp.dot(rc, t_v, preferred_element_type=jnp.float32)
        t_u = jnp.dot(ufb, wu_ref[k], preferred_element_type=jnp.float32)
        vcontribT = vcontribT + jax.lax.dot_general(
            t_u, rc, (((0,), (0,)), ((), ())),
            preferred_element_type=jnp.float32)

    code_ref[...] = jnp.where(valid, code_f, 0.0).astype(jnp.int8)

    @pl.when(j == 0)
    def _():
        u2_ref[...] = ucontrib

    @pl.when(j > 0)
    def _():
        u2_ref[...] = u2_ref[...] + ucontrib

    @pl.when(j == _GV - 1)
    def _():
        u2_ref[...] = jnp.maximum(u2_ref[...] + bu_ref[...], 0.0)

    sl = pl.ds(j * _BV, _BV)

    @pl.when(i == 0)
    def _():
        v2t_ref[:, sl] = vcontribT

    @pl.when(i > 0)
    def _():
        v2t_ref[:, sl] = v2t_ref[:, sl] + vcontribT

    @pl.when(i == _GU - 1)
    def _():
        v2t_ref[:, sl] = jnp.maximum(v2t_ref[:, sl] + bv_ref[...], 0.0)


def _decode_kernel(u2_ref, v2t_ref, q_ref, code_ref,
                   out_ref, loss_ref, acc_ref, sums):
    i = pl.program_id(0)
    j = pl.program_id(1)

    @pl.when(jnp.logical_and(i == 0, j == 0))
    def _():
        sums[0] = 0.0
        sums[1] = 0.0
        sums[2] = 0.0

    rows = jax.lax.broadcasted_iota(jnp.int32, (_BU, 1), 0) + i * _BU
    cols = jax.lax.broadcasted_iota(jnp.int32, (1, _BV), 1) + j * _BV
    valid = jnp.logical_and(rows < _NU, cols < _NV)

    u2b = u2_ref[...]
    v2tb = v2t_ref[...]
    code = code_ref[...].astype(jnp.int32)
    rated = jnp.logical_and(valid, code > 0)
    tcls = code - 1

    zs = []
    for k in range(_NC):
        uq = jnp.dot(u2b, q_ref[k], preferred_element_type=jnp.float32)
        z = jnp.dot(uq, v2tb, preferred_element_type=jnp.float32)
        out_ref[k] = z
        zs.append(z)

    m = zs[0]
    pred = jnp.zeros((_BU, _BV), jnp.int32)
    for k in range(1, _NC):
        gt = zs[k] > m
        pred = jnp.where(gt, k, pred)
        m = jnp.maximum(m, zs[k])
    s = jnp.zeros((_BU, _BV), jnp.float32)
    for k in range(_NC):
        s = s + jnp.exp(zs[k] - m)
    lse = m + jnp.log(s)

    ztrue = jnp.zeros((_BU, _BV), jnp.float32)
    for k in range(_NC):
        ztrue = jnp.where(tcls == k, zs[k], ztrue)

    loss_c = jnp.sum(jnp.where(rated, ztrue - lse, 0.0))
    mask_c = jnp.sum(jnp.where(rated, 1.0, 0.0))
    corr_c = jnp.sum(jnp.where(jnp.logical_and(rated, pred == tcls), 1.0, 0.0))
    sums[0] = sums[0] + loss_c
    sums[1] = sums[1] + mask_c
    sums[2] = sums[2] + corr_c

    @pl.when(jnp.logical_and(i == _GU - 1, j == _GV - 1))
    def _():
        denom = jnp.maximum(sums[1], 1.0)
        loss_ref[...] = jnp.full((1, 1), -sums[0] / denom, jnp.float32)
        acc_ref[...] = jnp.full((1, 1), sums[2] / denom, jnp.float32)


def kernel(u, v, r, n, c, u_emb_w, v_emb_w, Wu1, Wv1, bu1, bv1,
           Wu2, Wv2, bu2, bv2, Q):
    uf = jnp.take(u_emb_w, u, axis=0)
    vf = jnp.take(v_emb_w, v, axis=0)

    u2p, v2t, code = pl.pallas_call(
        _gconv_kernel,
        grid=(_GU, _GV),
        in_specs=[
            pl.BlockSpec((_NC, _BU, _BV), lambda i, j: (0, i, j)),
            pl.BlockSpec((_BU, _BV), lambda i, j: (i, j)),
            pl.BlockSpec((_BU, _D), lambda i, j: (i, 0)),
            pl.BlockSpec((_BV, _D), lambda i, j: (j, 0)),
            pl.BlockSpec((_NC, _D, _H), lambda i, j: (0, 0, 0)),
            pl.BlockSpec((_NC, _D, _H), lambda i, j: (0, 0, 0)),
            pl.BlockSpec((1, _H), lambda i, j: (0, 0)),
            pl.BlockSpec((_H, 1), lambda i, j: (0, 0)),
        ],
        out_specs=[
            pl.BlockSpec((_BU, _H), lambda i, j: (i, 0)),
            pl.BlockSpec((_H, _GV * _BV), lambda i, j: (0, 0)),
            pl.BlockSpec((_BU, _BV), lambda i, j: (i, j)),
        ],
        out_shape=[
            jax.ShapeDtypeStruct((_GU * _BU, _H), jnp.float32),
            jax.ShapeDtypeStruct((_H, _GV * _BV), jnp.float32),
            jax.ShapeDtypeStruct((_GU * _BU, _GV * _BV), jnp.int8),
        ],
        compiler_params=pltpu.CompilerParams(
            dimension_semantics=("arbitrary", "arbitrary")),
    )(r, c, uf, vf, Wu2, Wv2, bu2.reshape(1, _H), bv2.reshape(_H, 1))

    outputs, lossm, accm = pl.pallas_call(
        _decode_kernel,
        grid=(_GU, _GV),
        in_specs=[
            pl.BlockSpec((_BU, _H), lambda i, j: (i, 0)),
            pl.BlockSpec((_H, _BV), lambda i, j: (0, j)),
            pl.BlockSpec((_NC, _H, _H), lambda i, j: (0, 0, 0)),
            pl.BlockSpec((_BU, _BV), lambda i, j: (i, j)),
        ],
        out_specs=[
            pl.BlockSpec((_NC, _BU, _BV), lambda i, j: (0, i, j)),
            pl.BlockSpec((1, 1), lambda i, j: (0, 0)),
            pl.BlockSpec((1, 1), lambda i, j: (0, 0)),
        ],
        out_shape=[
            jax.ShapeDtypeStruct((_NC, _NU, _NV), jnp.float32),
            jax.ShapeDtypeStruct((1, 1), jnp.float32),
            jax.ShapeDtypeStruct((1, 1), jnp.float32),
        ],
        scratch_shapes=[pltpu.SMEM((4,), jnp.float32)],
        compiler_params=pltpu.CompilerParams(
            dimension_semantics=("arbitrary", "arbitrary")),
    )(u2p, v2t, Q, code)

    return outputs, lossm[0, 0], accm[0, 0]
